# baseline (speedup 1.0000x reference)
import functools

import jax
import jax.numpy as jnp
from jax.experimental import pallas as pl

TEMPERATURE = 10.0
TOP_FRACTION = 0.1
LAMBDA_TOPK = 50
IC_W = 0.5

_SUB = 8
_TILE_L = 56


def _softplus(x):
    return jnp.maximum(x, 0.0) + jnp.log1p(jnp.exp(-jnp.abs(x)))


def _xor_partner(x, bit_d, d, lanes):
    if d < lanes:
        axis, amt = 1, d
    else:
        axis, amt = 0, d // lanes
    return jnp.where(bit_d, jnp.roll(x, amt, axis=axis),
                     jnp.roll(x, -amt, axis=axis))


def _stage(arrs, i_lin, off, size, d, lanes):
    bit_d = ((i_lin + off) & d) != 0
    partners = [_xor_partner(a, bit_d, d, lanes) for a in arrs]
    v, ix = arrs[0], arrs[1]
    pv, pix = partners[0], partners[1]
    own_less = (v > pv) | ((v == pv) & (ix < pix))
    bit_s = ((i_lin + off) & size) != 0
    take_own = own_less == (bit_d == bit_s)
    return [jnp.where(take_own, a, p) for a, p in zip(arrs, partners)]


def _cross_stage(lo, hi):
    v, ix = lo[0], lo[1]
    pv, pix = hi[0], hi[1]
    lo_less = (v > pv) | ((v == pv) & (ix < pix))
    new_lo = [jnp.where(lo_less, a, b) for a, b in zip(lo, hi)]
    new_hi = [jnp.where(lo_less, b, a) for a, b in zip(lo, hi)]
    return new_lo, new_hi


def _fused_kernel(n, nc, k_pair, k_lam, yp_ref, yt_ref, out_ref):
    lanes = n // _SUB
    lanes_h = lanes // 2
    half = n // 2
    i_lin = (jax.lax.broadcasted_iota(
        jnp.int32, (_SUB, lanes_h), 0) * lanes_h
        + jax.lax.broadcasted_iota(jnp.int32, (_SUB, lanes_h), 1))
    yp8 = yp_ref[...]
    yt8 = yt_ref[...]
    yp_h = [yp8[:, 0:lanes_h], yp8[:, lanes_h:lanes]]
    yt_h = [yt8[:, 0:lanes_h], yt8[:, lanes_h:lanes]]

    s_pred = [[yp_h[h], i_lin + h * half, yt_h[h]] for h in (0, 1)]
    s_true = [[yt_h[h], i_lin + h * half, yp_h[h]] for h in (0, 1)]
    size = 2
    while size <= n:
        d = size // 2
        while d >= 1:
            if d == half:
                s_pred = list(_cross_stage(*s_pred))
                s_true = list(_cross_stage(*s_true))
            else:
                for h, off in ((0, 0), (1, half)):
                    s_pred[h] = _stage(s_pred[h], i_lin, off, size, d,
                                       lanes_h)
                    s_true[h] = _stage(s_true[h], i_lin, off, size, d,
                                       lanes_h)
            d //= 2
        size *= 2
    (yp_sp0, ix_sp0, yt_sp0), (yp_sp1, ix_sp1, yt_sp1) = s_pred
    (yt_bt0, ix_bt0, yp_bt0), (yt_bt1, ix_bt1, yp_bt1) = s_true

    mn = jnp.min(yt8)
    mx = jnp.max(yt8)
    denom = (mx - mn) + 1e-8
    disc0 = 1.0 / jnp.log2(i_lin.astype(jnp.float32) + 2.0)
    disc1 = 1.0 / jnp.log2(i_lin.astype(jnp.float32) + (2.0 + half))
    ideal = (jnp.sum(((yt_bt0 - mn) / denom) * disc0)
             + jnp.sum(((yt_bt1 - mn) / denom) * disc1) + 1e-8)

    a = yt8 * TEMPERATURE
    e = jnp.exp(a - jnp.max(a))
    p_true = e / jnp.sum(e)
    bm = jnp.max(yp8)
    logsm = (yp8 - bm) - jnp.log(jnp.sum(jnp.exp(yp8 - bm)))
    listnet = -jnp.sum(p_true * logsm)

    xc = yp8 - jnp.mean(yp8)
    yc = yt8 - jnp.mean(yt8)
    xs = jnp.sqrt(jnp.mean(xc * xc) + 1e-12)
    ys = jnp.sqrt(jnp.mean(yc * yc) + 1e-12)
    corr = jnp.clip(jnp.mean(xc * yc) / (xs * ys + 1e-12), -1.0, 1.0)

    rows = nc // lanes_h

    def _col(arr):
        return jnp.concatenate(
            [arr[r:r + 1, :].reshape(lanes_h, 1) for r in range(rows)],
            axis=0)

    ytc = _col(yt_bt0)
    ypc = _col(yp_bt0)
    ixc = _col(ix_bt0)

    yp_rl = ypc[0:_TILE_L].reshape(_TILE_L, 1, 1)
    yt_rl = ytc[0:_TILE_L].reshape(_TILE_L, 1, 1)
    g_rl = (yt_rl - mn) / denom
    ix_rl = ixc[0:_TILE_L].reshape(_TILE_L, 1, 1)
    top_lam = jax.lax.broadcasted_iota(jnp.int32, (_TILE_L, 1, 1), 0) < k_lam

    d_rl = 0.0
    for ix_sp, d3 in ((ix_sp0, disc0), (ix_sp1, disc1)):
        onehot = ix_rl == ix_sp[None]
        d_rl += jnp.sum(jnp.where(onehot, d3[None], 0.0), axis=(1, 2),
                        keepdims=True)

    lam_num = 0.0
    lam_cnt = 0.0
    for yp_sp, yt_sp, d3 in ((yp_sp0, yt_sp0, disc0),
                             (yp_sp1, yt_sp1, disc1)):
        yp3 = yp_sp[None]
        yt3 = yt_sp[None]
        g3 = (yt3 - mn) / denom
        xl = yp3 - yp_rl
        spl = _softplus(xl)
        tdl = yt_rl - yt3
        m_lam = (tdl > 0.0) & top_lam
        delta = jnp.abs((g_rl - g3) * (d_rl - d3[None]))
        lam_num += jnp.sum(jnp.where(m_lam, spl * delta, 0.0))
        lam_cnt += jnp.sum(m_lam.astype(jnp.float32))

    pair_num = 0.0
    pair_cnt = 0.0
    tile_r = nc // 2
    for r0 in range(0, nc, tile_r):
        yp_r = ypc[r0:r0 + tile_r].reshape(tile_r, 1, 1)
        yt_r = ytc[r0:r0 + tile_r].reshape(tile_r, 1, 1)
        x = yp8[None] - yp_r
        sp = _softplus(x)
        td = yt_r - yt8[None]
        m_pair = td > 0.0
        if r0 + tile_r > k_pair:
            gi = r0 + jax.lax.broadcasted_iota(jnp.int32, (tile_r, 1, 1), 0)
            m_pair = m_pair & (gi < k_pair)
        pair_num += jnp.sum(jnp.where(m_pair, sp * td, 0.0))
        pair_cnt += jnp.sum(m_pair.astype(jnp.float32))

    pair_loss = pair_num / (pair_cnt + 1e-8)
    lam_loss = jnp.where(
        lam_cnt > 0.0, (lam_num / ideal) / jnp.maximum(lam_cnt, 1.0), 0.0)
    total = listnet - IC_W * corr + pair_loss + lam_loss
    out_ref[...] = jnp.full((1, 1), total, dtype=jnp.float32)


def kernel(y_pred, y_true):
    n = y_pred.shape[1]
    k_pair = max(1, int(n * TOP_FRACTION))
    k_lam = min(LAMBDA_TOPK, n)
    nc = min(n, -(-k_pair // 256) * 256)
    lanes = n // _SUB
    assert k_lam <= _TILE_L <= nc and (n & (n - 1)) == 0 and nc % 16 == 0
    assert nc <= (n // 2) // 2

    yp8 = y_pred.reshape(_SUB, lanes)
    yt8 = y_true.reshape(_SUB, lanes)

    out = pl.pallas_call(
        functools.partial(_fused_kernel, n, nc, k_pair, k_lam),
        out_shape=jax.ShapeDtypeStruct((1, 1), jnp.float32),
    )(yp8, yt8)

    return out[0, 0]
